# baseline (speedup 1.0000x reference)
import functools

import jax
import jax.numpy as jnp
from jax import lax
from jax.experimental import pallas as pl
from jax.experimental.pallas import tpu as pltpu
from jax.experimental.pallas import tpu_sc as plsc

NW = 32
LANES = 16


def _sc_mesh():
    return plsc.VectorSubcoreMesh(core_axis_name="c", subcore_axis_name="s")


def _sc_params():
    return pltpu.CompilerParams(needs_layout_passes=False,
                                use_tc_tiling_on_sc=False)


def _scalar_gather_diff(table_pad, idx_flat, npad):
    npw = npad // NW
    gpw = npw // LANES
    epw = npw * 16

    @functools.partial(
        pl.kernel,
        out_type=jax.ShapeDtypeStruct((npad,), jnp.float32),
        mesh=_sc_mesh(),
        scratch_types=[
            pltpu.VMEM((npad,), jnp.float32),
            pltpu.VMEM((epw,), jnp.int32),
            pltpu.VMEM((npw,), jnp.float32),
        ],
        compiler_params=_sc_params(),
    )
    def k(tab_hbm, idx_hbm, out_hbm, tab_v, idx_v, out_v):
        wid = lax.axis_index("s") * 2 + lax.axis_index("c")
        pltpu.sync_copy(tab_hbm, tab_v)
        pltpu.sync_copy(idx_hbm.at[pl.ds(wid * epw, epw)], idx_v)
        lanes16 = lax.iota(jnp.int32, 16) * 16

        @pl.loop(0, gpw)
        def _(g):
            base = g * 256
            acc = None
            for kk in range(16):
                iv = plsc.load_gather(idx_v, [lanes16 + (base + kk)])
                v = plsc.load_gather(tab_v, [iv])
                acc = v if acc is None else acc + v
            own = tab_v[pl.ds(wid * npw + g * 16, 16)]
            out_v[pl.ds(g * 16, 16)] = 16.0 * own - acc

        pltpu.sync_copy(out_v, out_hbm.at[pl.ds(wid * npw, npw)])

    return k(table_pad, idx_flat)


def _row_gather_sum(table, idx2, npad):
    npw = npad // NW
    nchunk = npw // 32
    assert nchunk % 2 == 0

    @functools.partial(
        pl.kernel,
        out_type=jax.ShapeDtypeStruct((npad, 64), jnp.bfloat16),
        mesh=_sc_mesh(),
        scratch_types=[
            pltpu.VMEM((2, 4, 128), jnp.int32),
            pltpu.VMEM((2, 512, 64), jnp.bfloat16),
            pltpu.VMEM((32, 64), jnp.bfloat16),
            pltpu.SemaphoreType.DMA,
            pltpu.SemaphoreType.DMA,
        ],
        compiler_params=_sc_params(),
    )
    def k(tab_hbm, idx_hbm, out_hbm, idx_v, rows_v, red_v, sem0, sem1):
        wid = lax.axis_index("s") * 2 + lax.axis_index("c")
        idx_row0 = wid * (nchunk * 4)
        sems = (sem0, sem1)

        def fire(buf, m):
            pltpu.sync_copy(idx_hbm.at[pl.ds(idx_row0 + m * 4, 4)],
                            idx_v.at[buf])
            for j in range(4):
                pltpu.async_copy(tab_hbm.at[idx_v.at[buf].at[j]],
                                 rows_v.at[buf].at[pl.ds(j * 128, 128)],
                                 sems[buf])

        def drain(buf):
            pltpu.make_async_copy(tab_hbm.at[pl.ds(0, 512)],
                                  rows_v.at[buf], sems[buf]).wait()

        fire(0, 0)

        @pl.loop(0, nchunk // 2)
        def _(m2):
            for buf in range(2):
                m = m2 * 2 + buf
                drain(buf)
                if buf == 0:
                    fire(1, m + 1)
                else:
                    @pl.when(m2 < nchunk // 2 - 1)
                    def _():
                        fire(0, m + 1)

                @pl.loop(0, 32)
                def _(w):
                    for c in range(2):
                        evs, ods = [], []
                        for kk in range(16):
                            x = rows_v[buf, w * 16 + kk, pl.ds(c * 32, 32)]
                            e, o = plsc.unpack(
                                x, format=plsc.PackFormat.INTERLEAVED)
                            evs.append(e)
                            ods.append(o)
                        while len(evs) > 1:
                            evs = [evs[2 * i] + evs[2 * i + 1]
                                   for i in range(len(evs) // 2)]
                            ods = [ods[2 * i] + ods[2 * i + 1]
                                   for i in range(len(ods) // 2)]
                        red_v[w, pl.ds(c * 32, 32)] = plsc.pack(
                            evs[0], ods[0],
                            format=plsc.PackFormat.INTERLEAVED)

                pltpu.sync_copy(red_v,
                                out_hbm.at[pl.ds(wid * npw + m * 32, 32)])

    return k(table, idx2)


def _dense_pre(r80, wc, bc, n, bn):
    def body(r_ref, w_ref, b_ref, o1, o2, o3):
        c = jnp.dot(r_ref[...], w_ref[...],
                    preferred_element_type=jnp.float32,
                    precision=lax.Precision.HIGHEST)
        c = c + 16.0 * b_ref[...]
        o1[...] = c[:, 0:64]
        o2[...] = c[:, 64:128]
        o3[...] = c[:, 128:129]

    return pl.pallas_call(
        body,
        grid=(n // bn,),
        in_specs=[pl.BlockSpec((bn, 80), lambda i: (i, 0)),
                  pl.BlockSpec((80, 129), lambda i: (0, 0)),
                  pl.BlockSpec((1, 129), lambda i: (0, 0))],
        out_specs=[pl.BlockSpec((bn, 64), lambda i: (i, 0)),
                   pl.BlockSpec((bn, 64), lambda i: (i, 0)),
                   pl.BlockSpec((bn, 1), lambda i: (i, 0))],
        out_shape=[jax.ShapeDtypeStruct((n, 64), jnp.float32),
                   jax.ShapeDtypeStruct((n, 64), jnp.float32),
                   jax.ShapeDtypeStruct((n, 1), jnp.float32)],
    )(r80, wc, bc)


def _dense_l1(c1, d1, w1d, w2d, n, bn):
    def body(c1_ref, d1_ref, w1_ref, w2_ref, o_ref, oh_ref):
        f1 = jnp.maximum(c1_ref[...] + d1_ref[...] * w1_ref[...], 0.0)
        t1 = jnp.dot(f1, w2_ref[...],
                     preferred_element_type=jnp.float32,
                     precision=lax.Precision.HIGHEST)
        o_ref[...] = t1
        oh_ref[...] = t1.astype(jnp.bfloat16)

    return pl.pallas_call(
        body,
        grid=(n // bn,),
        in_specs=[pl.BlockSpec((bn, 64), lambda i: (i, 0)),
                  pl.BlockSpec((bn, 1), lambda i: (i, 0)),
                  pl.BlockSpec((1, 64), lambda i: (0, 0)),
                  pl.BlockSpec((64, 64), lambda i: (0, 0))],
        out_specs=[pl.BlockSpec((bn, 64), lambda i: (i, 0)),
                   pl.BlockSpec((bn, 64), lambda i: (i, 0))],
        out_shape=[jax.ShapeDtypeStruct((n, 64), jnp.float32),
                   jax.ShapeDtypeStruct((n, 64), jnp.bfloat16)],
    )(c1, d1, w1d, w2d)


def _dense_l2(c2, t1, gs, w3d, n, bn):
    def body(c2_ref, t1_ref, g_ref, w3_ref, o_ref):
        f2 = jnp.maximum(c2_ref[...] + 16.0 * t1_ref[...]
                         - g_ref[...].astype(jnp.float32), 0.0)
        o_ref[...] = jnp.dot(f2, w3_ref[...],
                             preferred_element_type=jnp.float32,
                             precision=lax.Precision.HIGHEST)

    return pl.pallas_call(
        body,
        grid=(n // bn,),
        in_specs=[pl.BlockSpec((bn, 64), lambda i: (i, 0)),
                  pl.BlockSpec((bn, 64), lambda i: (i, 0)),
                  pl.BlockSpec((bn, 64), lambda i: (i, 0)),
                  pl.BlockSpec((64, 1), lambda i: (0, 0))],
        out_specs=pl.BlockSpec((bn, 1), lambda i: (i, 0)),
        out_shape=jax.ShapeDtypeStruct((n, 1), jnp.float32),
    )(c2, t1, gs, w3d)


def _softmax_out(c3, d3, rows, cols):
    def body(c3_ref, d3_ref, o_ref):
        x = c3_ref[...] + d3_ref[...]
        e = jnp.exp(x - jnp.max(x))
        o_ref[...] = e / jnp.sum(e)

    return pl.pallas_call(
        body,
        out_shape=jax.ShapeDtypeStruct((rows, cols), jnp.float32),
    )(c3, d3)


def kernel(p_init, r_matrix, indices_neigh_tri, W1, b1, W2, b2, W3, b3):
    n, kp1 = indices_neigh_tri.shape
    kk = kp1 - 1
    r = r_matrix.shape[2]
    h = W1.shape[1]
    assert kk == 16 and r == 5 and h == 64
    npad = ((n + 2047) // 2048) * 2048
    bn = 2000
    assert n % bn == 0

    neigh = indices_neigh_tri[:, 1:].astype(jnp.int32)
    neigh_p = jnp.pad(neigh, ((0, npad - n), (0, 0)))
    idx2 = neigh_p.reshape(-1, 128)
    idx_flat = neigh_p.reshape(-1)

    r80 = r_matrix.reshape(n, kk * r)
    wc = jnp.concatenate([jnp.tile(W1[:r], (kk, 1)),
                          jnp.tile(W2[:r], (kk, 1)),
                          jnp.tile(W3[:r], (kk, 1))], axis=1)
    bc = jnp.concatenate([b1, b2, b3])[None, :]

    c1, c2, c3 = _dense_pre(r80, wc, bc, n, bn)
    p_pad = jnp.pad(p_init, (0, npad - n))
    d1 = _scalar_gather_diff(p_pad, idx_flat, npad)[:n].reshape(n, 1)
    t1, t1h = _dense_l1(c1, d1, W1[r:r + 1], W2[r:], n, bn)
    gs = _row_gather_sum(t1h, idx2, npad)
    g = _dense_l2(c2, t1, gs, W3[r:], n, bn)
    g_pad = jnp.pad(g.reshape(n), (0, npad - n))
    d3 = _scalar_gather_diff(g_pad, idx_flat, npad)[:n]
    rows, cols = 400, n // 400
    return _softmax_out(c3.reshape(rows, cols), d3.reshape(rows, cols),
                        rows, cols).reshape(n)


# speedup vs baseline: 1.2639x; 1.2639x over previous
import functools

import jax
import jax.numpy as jnp
from jax import lax
from jax.experimental import pallas as pl
from jax.experimental.pallas import tpu as pltpu
from jax.experimental.pallas import tpu_sc as plsc

NW = 32
LANES = 16


def _sc_mesh():
    return plsc.VectorSubcoreMesh(core_axis_name="c", subcore_axis_name="s")


def _sc_params():
    return pltpu.CompilerParams(needs_layout_passes=False,
                                use_tc_tiling_on_sc=False)


def _scalar_gather_diff(table_pad, idx_flat, npad):
    npw = npad // NW
    gpw = npw // LANES
    epw = npw * 16

    @functools.partial(
        pl.kernel,
        out_type=jax.ShapeDtypeStruct((npad,), jnp.float32),
        mesh=_sc_mesh(),
        scratch_types=[
            pltpu.VMEM((npad,), jnp.float32),
            pltpu.VMEM((epw,), jnp.int32),
            pltpu.VMEM((npw,), jnp.float32),
        ],
        compiler_params=_sc_params(),
    )
    def k(tab_hbm, idx_hbm, out_hbm, tab_v, idx_v, out_v):
        wid = lax.axis_index("s") * 2 + lax.axis_index("c")
        pltpu.sync_copy(tab_hbm, tab_v)
        pltpu.sync_copy(idx_hbm.at[pl.ds(wid * epw, epw)], idx_v)
        lanes16 = lax.iota(jnp.int32, 16) * 16

        @pl.loop(0, gpw)
        def _(g):
            base = g * 256
            acc = None
            for kk in range(16):
                iv = plsc.load_gather(idx_v, [lanes16 + (base + kk)])
                v = plsc.load_gather(tab_v, [iv])
                acc = v if acc is None else acc + v
            own = tab_v[pl.ds(wid * npw + g * 16, 16)]
            out_v[pl.ds(g * 16, 16)] = 16.0 * own - acc

        pltpu.sync_copy(out_v, out_hbm.at[pl.ds(wid * npw, npw)])

    return k(table_pad, idx_flat)


def _row_gather_sum(tab0, tab1, idx2, npad):
    npt = npad // 16
    nchunk = npt // 32
    assert nchunk % 2 == 0
    ntab = tab0.shape[0]
    tpw = ntab // 16

    @functools.partial(
        pl.kernel,
        out_type=jax.ShapeDtypeStruct((2, npad, 32), jnp.bfloat16),
        mesh=_sc_mesh(),
        scratch_types=[
            pltpu.VMEM((2, 4, 128), jnp.int32),
            pltpu.VMEM((2, 512, 32), jnp.bfloat16),
            pltpu.VMEM((32, 32), jnp.bfloat16),
            pltpu.VMEM_SHARED((ntab, 32), jnp.bfloat16),
            pltpu.SemaphoreType.DMA,
            pltpu.SemaphoreType.DMA,
        ],
        compiler_params=_sc_params(),
    )
    def k(tab0_hbm, tab1_hbm, idx_hbm, out_hbm, idx_v, rows_v, red_v,
          tab_sh, sem0, sem1):
        cid = lax.axis_index("c")
        sid = lax.axis_index("s")
        idx_row0 = sid * (nchunk * 4)
        sems = (sem0, sem1)

        @pl.when(cid == 0)
        def _():
            pltpu.sync_copy(tab0_hbm.at[pl.ds(sid * tpw, tpw)],
                            tab_sh.at[pl.ds(sid * tpw, tpw)])

        @pl.when(cid == 1)
        def _():
            pltpu.sync_copy(tab1_hbm.at[pl.ds(sid * tpw, tpw)],
                            tab_sh.at[pl.ds(sid * tpw, tpw)])

        plsc.subcore_barrier()

        def fire(buf, m):
            pltpu.sync_copy(idx_hbm.at[pl.ds(idx_row0 + m * 4, 4)],
                            idx_v.at[buf])
            for j in range(4):
                pltpu.async_copy(tab_sh.at[idx_v.at[buf].at[j]],
                                 rows_v.at[buf].at[pl.ds(j * 128, 128)],
                                 sems[buf])

        def drain(buf):
            pltpu.make_async_copy(tab0_hbm.at[pl.ds(0, 512)],
                                  rows_v.at[buf], sems[buf]).wait()

        fire(0, 0)

        @pl.loop(0, nchunk // 2)
        def _(m2):
            for buf in range(2):
                m = m2 * 2 + buf
                drain(buf)
                if buf == 0:
                    fire(1, m + 1)
                else:
                    @pl.when(m2 < nchunk // 2 - 1)
                    def _():
                        fire(0, m + 1)

                @pl.loop(0, 32)
                def _(w):
                    evs, ods = [], []
                    for kk in range(16):
                        x = rows_v[buf, w * 16 + kk, :]
                        e, o = plsc.unpack(
                            x, format=plsc.PackFormat.INTERLEAVED)
                        evs.append(e)
                        ods.append(o)
                    while len(evs) > 1:
                        evs = [evs[2 * i] + evs[2 * i + 1]
                               for i in range(len(evs) // 2)]
                        ods = [ods[2 * i] + ods[2 * i + 1]
                               for i in range(len(ods) // 2)]
                    red_v[w, :] = plsc.pack(
                        evs[0], ods[0], format=plsc.PackFormat.INTERLEAVED)

                pltpu.sync_copy(
                    red_v,
                    out_hbm.at[cid].at[pl.ds(sid * npt + m * 32, 32)])

    return k(tab0, tab1, idx2)


def _dense_pre(r80, wc, bc, n, bn):
    def body(r_ref, w_ref, b_ref, o1, o2, o3):
        c = jnp.dot(r_ref[...], w_ref[...],
                    preferred_element_type=jnp.float32,
                    precision=lax.Precision.HIGHEST)
        c = c + 16.0 * b_ref[...]
        o1[...] = c[:, 0:64]
        o2[...] = c[:, 64:128]
        o3[...] = c[:, 128:129]

    return pl.pallas_call(
        body,
        grid=(n // bn,),
        in_specs=[pl.BlockSpec((bn, 80), lambda i: (i, 0)),
                  pl.BlockSpec((80, 129), lambda i: (0, 0)),
                  pl.BlockSpec((1, 129), lambda i: (0, 0))],
        out_specs=[pl.BlockSpec((bn, 64), lambda i: (i, 0)),
                   pl.BlockSpec((bn, 64), lambda i: (i, 0)),
                   pl.BlockSpec((bn, 1), lambda i: (i, 0))],
        out_shape=[jax.ShapeDtypeStruct((n, 64), jnp.float32),
                   jax.ShapeDtypeStruct((n, 64), jnp.float32),
                   jax.ShapeDtypeStruct((n, 1), jnp.float32)],
    )(r80, wc, bc)


def _dense_l1(c1, d1, w1d, w2d, n, bn):
    def body(c1_ref, d1_ref, w1_ref, w2_ref, o_ref, oh0_ref, oh1_ref):
        f1 = jnp.maximum(c1_ref[...] + d1_ref[...] * w1_ref[...], 0.0)
        t1 = jnp.dot(f1, w2_ref[...],
                     preferred_element_type=jnp.float32,
                     precision=lax.Precision.HIGHEST)
        o_ref[...] = t1
        th = t1.astype(jnp.bfloat16)
        oh0_ref[...] = th[:, 0:32]
        oh1_ref[...] = th[:, 32:64]

    return pl.pallas_call(
        body,
        grid=(n // bn,),
        in_specs=[pl.BlockSpec((bn, 64), lambda i: (i, 0)),
                  pl.BlockSpec((bn, 1), lambda i: (i, 0)),
                  pl.BlockSpec((1, 64), lambda i: (0, 0)),
                  pl.BlockSpec((64, 64), lambda i: (0, 0))],
        out_specs=[pl.BlockSpec((bn, 64), lambda i: (i, 0)),
                   pl.BlockSpec((bn, 32), lambda i: (i, 0)),
                   pl.BlockSpec((bn, 32), lambda i: (i, 0))],
        out_shape=[jax.ShapeDtypeStruct((n, 64), jnp.float32),
                   jax.ShapeDtypeStruct((n, 32), jnp.bfloat16),
                   jax.ShapeDtypeStruct((n, 32), jnp.bfloat16)],
    )(c1, d1, w1d, w2d)


def _dense_l2(c2, t1, gs0, gs1, w3d, n, bn):
    def body(c2_ref, t1_ref, g0_ref, g1_ref, w3_ref, o_ref):
        g = jnp.concatenate([g0_ref[...], g1_ref[...]],
                            axis=1).astype(jnp.float32)
        f2 = jnp.maximum(c2_ref[...] + 16.0 * t1_ref[...] - g, 0.0)
        o_ref[...] = jnp.dot(f2, w3_ref[...],
                             preferred_element_type=jnp.float32,
                             precision=lax.Precision.HIGHEST)

    return pl.pallas_call(
        body,
        grid=(n // bn,),
        in_specs=[pl.BlockSpec((bn, 64), lambda i: (i, 0)),
                  pl.BlockSpec((bn, 64), lambda i: (i, 0)),
                  pl.BlockSpec((bn, 32), lambda i: (i, 0)),
                  pl.BlockSpec((bn, 32), lambda i: (i, 0)),
                  pl.BlockSpec((64, 1), lambda i: (0, 0))],
        out_specs=pl.BlockSpec((bn, 1), lambda i: (i, 0)),
        out_shape=jax.ShapeDtypeStruct((n, 1), jnp.float32),
    )(c2, t1, gs0, gs1, w3d)


def _softmax_out(c3, d3, rows, cols):
    def body(c3_ref, d3_ref, o_ref):
        x = c3_ref[...] + d3_ref[...]
        e = jnp.exp(x - jnp.max(x))
        o_ref[...] = e / jnp.sum(e)

    return pl.pallas_call(
        body,
        out_shape=jax.ShapeDtypeStruct((rows, cols), jnp.float32),
    )(c3, d3)


def kernel(p_init, r_matrix, indices_neigh_tri, W1, b1, W2, b2, W3, b3):
    n, kp1 = indices_neigh_tri.shape
    kk = kp1 - 1
    r = r_matrix.shape[2]
    h = W1.shape[1]
    assert kk == 16 and r == 5 and h == 64
    npad = ((n + 2047) // 2048) * 2048
    bn = 2000
    assert n % bn == 0

    neigh = indices_neigh_tri[:, 1:].astype(jnp.int32)
    neigh_p = jnp.pad(neigh, ((0, npad - n), (0, 0)))
    idx2 = neigh_p.reshape(-1, 128)
    idx_flat = neigh_p.reshape(-1)

    r80 = r_matrix.reshape(n, kk * r)
    wc = jnp.concatenate([jnp.tile(W1[:r], (kk, 1)),
                          jnp.tile(W2[:r], (kk, 1)),
                          jnp.tile(W3[:r], (kk, 1))], axis=1)
    bc = jnp.concatenate([b1, b2, b3])[None, :]

    c1, c2, c3 = _dense_pre(r80, wc, bc, n, bn)
    p_pad = jnp.pad(p_init, (0, npad - n))
    d1 = _scalar_gather_diff(p_pad, idx_flat, npad)[:n].reshape(n, 1)
    t1, th0, th1 = _dense_l1(c1, d1, W1[r:r + 1], W2[r:], n, bn)
    gs = _row_gather_sum(th0, th1, idx2, npad)
    g = _dense_l2(c2, t1, gs[0], gs[1], W3[r:], n, bn)
    g_pad = jnp.pad(g.reshape(n), (0, npad - n))
    d3 = _scalar_gather_diff(g_pad, idx_flat, npad)[:n]
    rows, cols = 400, n // 400
    return _softmax_out(c3.reshape(rows, cols), d3.reshape(rows, cols),
                        rows, cols).reshape(n)


# speedup vs baseline: 1.5800x; 1.2501x over previous
import functools

import jax
import jax.numpy as jnp
from jax import lax
from jax.experimental import pallas as pl
from jax.experimental.pallas import tpu as pltpu
from jax.experimental.pallas import tpu_sc as plsc

NW = 32
LANES = 16


def _sc_mesh():
    return plsc.VectorSubcoreMesh(core_axis_name="c", subcore_axis_name="s")


def _sc_params():
    return pltpu.CompilerParams(needs_layout_passes=False,
                                use_tc_tiling_on_sc=False)


def _scalar_gather_diff(table_pad, idx_flat, npad):
    npw = npad // NW
    gpw = npw // LANES
    epw = npw * 16

    @functools.partial(
        pl.kernel,
        out_type=jax.ShapeDtypeStruct((npad,), jnp.float32),
        mesh=_sc_mesh(),
        scratch_types=[
            pltpu.VMEM((npad,), jnp.float32),
            pltpu.VMEM((epw,), jnp.int32),
            pltpu.VMEM((npw,), jnp.float32),
        ],
        compiler_params=_sc_params(),
    )
    def k(tab_hbm, idx_hbm, out_hbm, tab_v, idx_v, out_v):
        wid = lax.axis_index("s") * 2 + lax.axis_index("c")
        pltpu.sync_copy(tab_hbm, tab_v)
        pltpu.sync_copy(idx_hbm.at[pl.ds(wid * epw, epw)], idx_v)
        lanes16 = lax.iota(jnp.int32, 16) * 16

        @pl.loop(0, gpw)
        def _(g):
            base = g * 256
            acc = None
            for kk in range(16):
                iv = plsc.load_gather(idx_v, [lanes16 + (base + kk)])
                v = plsc.load_gather(tab_v, [iv])
                acc = v if acc is None else acc + v
            own = tab_v[pl.ds(wid * npw + g * 16, 16)]
            out_v[pl.ds(g * 16, 16)] = 16.0 * own - acc

        pltpu.sync_copy(out_v, out_hbm.at[pl.ds(wid * npw, npw)])

    return k(table_pad, idx_flat)


def _row_gather_partial(tab0, tab1, yh0, yh1, idx2, w3eo, npad):
    npt = npad // 16
    nchunk = npt // 32
    assert nchunk % 2 == 0
    ntab = tab0.shape[0]
    tpw = ntab // 16
    yfull = ntab // npt
    ytail = ntab - yfull * npt

    @functools.partial(
        pl.kernel,
        out_type=jax.ShapeDtypeStruct((2, npad), jnp.float32),
        mesh=_sc_mesh(),
        scratch_types=[
            pltpu.VMEM((2, 4, 128), jnp.int32),
            pltpu.VMEM((2, 512, 32), jnp.bfloat16),
            pltpu.VMEM((npt, 32), jnp.bfloat16),
            pltpu.VMEM((npt,), jnp.float32),
            pltpu.VMEM((2, 32), jnp.float32),
            pltpu.VMEM_SHARED((ntab, 32), jnp.bfloat16),
            pltpu.SemaphoreType.DMA,
            pltpu.SemaphoreType.DMA,
        ],
        compiler_params=_sc_params(),
    )
    def k(tab0_hbm, tab1_hbm, yh0_hbm, yh1_hbm, idx_hbm, w3_hbm, out_hbm,
          idx_v, rows_v, y_v, gp_v, w3_v, tab_sh, sem0, sem1):
        cid = lax.axis_index("c")
        sid = lax.axis_index("s")
        idx_row0 = sid * (nchunk * 4)
        sems = (sem0, sem1)

        for cc, tab_h, y_h in ((0, tab0_hbm, yh0_hbm), (1, tab1_hbm, yh1_hbm)):
            @pl.when(cid == cc)
            def _():
                pltpu.sync_copy(tab_h.at[pl.ds(sid * tpw, tpw)],
                                tab_sh.at[pl.ds(sid * tpw, tpw)])

            @pl.when(jnp.logical_and(cid == cc, sid < yfull))
            def _():
                pltpu.sync_copy(y_h.at[pl.ds(sid * npt, npt)], y_v)

            if ytail > 0:
                @pl.when(jnp.logical_and(cid == cc, sid == yfull))
                def _():
                    pltpu.sync_copy(y_h.at[pl.ds(yfull * npt, ytail)],
                                    y_v.at[pl.ds(0, ytail)])

        pltpu.sync_copy(w3_hbm, w3_v)
        plsc.subcore_barrier()
        w3a = w3_v[cid, pl.ds(0, 16)]
        w3b = w3_v[cid, pl.ds(16, 16)]
        lanes_i = lax.iota(jnp.int32, 16)

        def fire(buf, m):
            pltpu.sync_copy(idx_hbm.at[pl.ds(idx_row0 + m * 4, 4)],
                            idx_v.at[buf])
            for j in range(4):
                pltpu.async_copy(tab_sh.at[idx_v.at[buf].at[j]],
                                 rows_v.at[buf].at[pl.ds(j * 128, 128)],
                                 sems[buf])

        def drain(buf):
            pltpu.make_async_copy(tab0_hbm.at[pl.ds(0, 512)],
                                  rows_v.at[buf], sems[buf]).wait()

        fire(0, 0)

        @pl.loop(0, nchunk // 2)
        def _(m2):
            for buf in range(2):
                m = m2 * 2 + buf
                drain(buf)
                if buf == 0:
                    fire(1, m + 1)
                else:
                    @pl.when(m2 < nchunk // 2 - 1)
                    def _():
                        fire(0, m + 1)

                @pl.loop(0, 2)
                def _(hh):
                    acc = jnp.zeros((16,), jnp.float32)
                    for w16 in range(16):
                        w = hh * 16 + w16
                        evs, ods = [], []
                        for kk in range(16):
                            x = rows_v[buf, w * 16 + kk, :]
                            e, o = plsc.unpack(
                                x, format=plsc.PackFormat.INTERLEAVED)
                            evs.append(e)
                            ods.append(o)
                        while len(evs) > 1:
                            evs = [evs[2 * i] + evs[2 * i + 1]
                                   for i in range(len(evs) // 2)]
                            ods = [ods[2 * i] + ods[2 * i + 1]
                                   for i in range(len(ods) // 2)]
                        ye, yo = plsc.unpack(
                            y_v[m * 32 + w, :],
                            format=plsc.PackFormat.INTERLEAVED)
                        fe = jnp.maximum(
                            ye.astype(jnp.float32) - evs[0], 0.0)
                        fo = jnp.maximum(
                            yo.astype(jnp.float32) - ods[0], 0.0)
                        s = jnp.sum(fe * w3a + fo * w3b)
                        acc = jnp.where(lanes_i == w16, s, acc)
                    gp_v[pl.ds(m * 32 + hh * 16, 16)] = acc

        pltpu.sync_copy(gp_v, out_hbm.at[cid].at[pl.ds(sid * npt, npt)])

    return k(tab0, tab1, yh0, yh1, idx2, w3eo)


def _dense_pre(r80, wc, bc, npad, bn):
    def body(r_ref, w_ref, b_ref, o1, o2, o3):
        c = jnp.dot(r_ref[...], w_ref[...],
                    preferred_element_type=jnp.float32,
                    precision=lax.Precision.HIGHEST)
        c = c + 16.0 * b_ref[...]
        o1[...] = c[:, 0:64]
        o2[...] = c[:, 64:128]
        o3[...] = c[:, 128:129]

    return pl.pallas_call(
        body,
        grid=(npad // bn,),
        in_specs=[pl.BlockSpec((bn, 80), lambda i: (i, 0)),
                  pl.BlockSpec((80, 129), lambda i: (0, 0)),
                  pl.BlockSpec((1, 129), lambda i: (0, 0))],
        out_specs=[pl.BlockSpec((bn, 64), lambda i: (i, 0)),
                   pl.BlockSpec((bn, 64), lambda i: (i, 0)),
                   pl.BlockSpec((bn, 1), lambda i: (i, 0))],
        out_shape=[jax.ShapeDtypeStruct((npad, 64), jnp.float32),
                   jax.ShapeDtypeStruct((npad, 64), jnp.float32),
                   jax.ShapeDtypeStruct((npad, 1), jnp.float32)],
    )(r80, wc, bc)


def _dense_l1(c1, c2, d1, w1d, w2d, npad, bn):
    def body(c1_ref, c2_ref, d1_ref, w1_ref, w2_ref,
             t0_ref, t1_ref, y0_ref, y1_ref):
        d1v = d1_ref[...].reshape(bn, 1)
        f1 = jnp.maximum(c1_ref[...] + d1v * w1_ref[...], 0.0)
        t1 = jnp.dot(f1, w2_ref[...],
                     preferred_element_type=jnp.float32,
                     precision=lax.Precision.HIGHEST)
        th = t1.astype(jnp.bfloat16)
        t0_ref[...] = th[:, 0:32]
        t1_ref[...] = th[:, 32:64]
        yh = (c2_ref[...] + 16.0 * t1).astype(jnp.bfloat16)
        y0_ref[...] = yh[:, 0:32]
        y1_ref[...] = yh[:, 32:64]

    return pl.pallas_call(
        body,
        grid=(npad // bn,),
        in_specs=[pl.BlockSpec((bn, 64), lambda i: (i, 0)),
                  pl.BlockSpec((bn, 64), lambda i: (i, 0)),
                  pl.BlockSpec((bn,), lambda i: (i,)),
                  pl.BlockSpec((1, 64), lambda i: (0, 0)),
                  pl.BlockSpec((64, 64), lambda i: (0, 0))],
        out_specs=[pl.BlockSpec((bn, 32), lambda i: (i, 0)),
                   pl.BlockSpec((bn, 32), lambda i: (i, 0)),
                   pl.BlockSpec((bn, 32), lambda i: (i, 0)),
                   pl.BlockSpec((bn, 32), lambda i: (i, 0))],
        out_shape=[jax.ShapeDtypeStruct((npad, 32), jnp.bfloat16),
                   jax.ShapeDtypeStruct((npad, 32), jnp.bfloat16),
                   jax.ShapeDtypeStruct((npad, 32), jnp.bfloat16),
                   jax.ShapeDtypeStruct((npad, 32), jnp.bfloat16)],
    )(c1, c2, d1, w1d, w2d)


def _softmax_out(c3, d3, rows, cols):
    def body(c3_ref, d3_ref, o_ref):
        x = c3_ref[...] + d3_ref[...]
        e = jnp.exp(x - jnp.max(x))
        o_ref[...] = e / jnp.sum(e)

    return pl.pallas_call(
        body,
        out_shape=jax.ShapeDtypeStruct((rows, cols), jnp.float32),
    )(c3, d3)


def kernel(p_init, r_matrix, indices_neigh_tri, W1, b1, W2, b2, W3, b3):
    n, kp1 = indices_neigh_tri.shape
    kk = kp1 - 1
    r = r_matrix.shape[2]
    h = W1.shape[1]
    assert kk == 16 and r == 5 and h == 64
    npad = ((n + 2047) // 2048) * 2048
    bn = 2048

    neigh = indices_neigh_tri[:, 1:].astype(jnp.int32)
    neigh_p = jnp.pad(neigh, ((0, npad - n), (0, 0)))
    idx2 = neigh_p.reshape(-1, 128)
    idx_flat = neigh_p.reshape(-1)

    r80 = r_matrix.reshape(n, kk * r)
    wc = jnp.concatenate([jnp.tile(W1[:r], (kk, 1)),
                          jnp.tile(W2[:r], (kk, 1)),
                          jnp.tile(W3[:r], (kk, 1))], axis=1)
    bc = jnp.concatenate([b1, b2, b3])[None, :]

    c1, c2, c3 = _dense_pre(r80, wc, bc, npad, bn)
    p_pad = jnp.pad(p_init, (0, npad - n))
    d1 = _scalar_gather_diff(p_pad, idx_flat, npad)
    th0, th1, yh0, yh1 = _dense_l1(c1, c2, d1, W1[r:r + 1], W2[r:],
                                   npad, bn)
    w3f = W3[r:, 0]
    w3eo = jnp.stack([
        jnp.concatenate([w3f[0:32:2], w3f[1:32:2]]),
        jnp.concatenate([w3f[32:64:2], w3f[33:64:2]])])
    gp = _row_gather_partial(th0, th1, yh0, yh1, idx2, w3eo, npad)
    g = gp[0] + gp[1]
    d3 = _scalar_gather_diff(g, idx_flat, npad)[:n]
    rows, cols = 400, n // 400
    return _softmax_out(c3[:n].reshape(rows, cols), d3.reshape(rows, cols),
                        rows, cols).reshape(n)


# speedup vs baseline: 1.6595x; 1.0503x over previous
import functools

import jax
import jax.numpy as jnp
from jax import lax
from jax.experimental import pallas as pl
from jax.experimental.pallas import tpu as pltpu
from jax.experimental.pallas import tpu_sc as plsc

NW = 32
LANES = 16


def _sc_mesh():
    return plsc.VectorSubcoreMesh(core_axis_name="c", subcore_axis_name="s")


def _sc_params():
    return pltpu.CompilerParams(needs_layout_passes=False,
                                use_tc_tiling_on_sc=False)


def _scalar_gather_diff(table_pad, idx_flat, npad):
    npw = npad // NW
    gpw = npw // LANES
    epw = npw * 16

    @functools.partial(
        pl.kernel,
        out_type=jax.ShapeDtypeStruct((npad,), jnp.float32),
        mesh=_sc_mesh(),
        scratch_types=[
            pltpu.VMEM((npad,), jnp.float32),
            pltpu.VMEM((epw,), jnp.int32),
            pltpu.VMEM((npw,), jnp.float32),
        ],
        compiler_params=_sc_params(),
    )
    def k(tab_hbm, idx_hbm, out_hbm, tab_v, idx_v, out_v):
        wid = lax.axis_index("s") * 2 + lax.axis_index("c")
        pltpu.sync_copy(tab_hbm, tab_v)
        pltpu.sync_copy(idx_hbm.at[pl.ds(wid * epw, epw)], idx_v)
        lanes16 = lax.iota(jnp.int32, 16) * 16

        @pl.loop(0, gpw)
        def _(g):
            base = g * 256
            acc = None
            for kk in range(16):
                iv = plsc.load_gather(idx_v, [lanes16 + (base + kk)])
                v = plsc.load_gather(tab_v, [iv])
                acc = v if acc is None else acc + v
            own = tab_v[pl.ds(wid * npw + g * 16, 16)]
            out_v[pl.ds(g * 16, 16)] = 16.0 * own - acc

        pltpu.sync_copy(out_v, out_hbm.at[pl.ds(wid * npw, npw)])

    return k(table_pad, idx_flat)


def _row_gather_partial(tab0, tab1, yh0, yh1, idx2, w3eo, npad):
    npt = npad // 16
    nchunk = npt // 32
    assert nchunk % 2 == 0
    ntab = tab0.shape[0]
    tpw = ntab // 16
    yfull = ntab // npt
    ytail = ntab - yfull * npt

    @functools.partial(
        pl.kernel,
        out_type=jax.ShapeDtypeStruct((2, npad), jnp.float32),
        mesh=_sc_mesh(),
        scratch_types=[
            pltpu.VMEM((2, 4, 128), jnp.int32),
            pltpu.VMEM((2, 512, 32), jnp.bfloat16),
            pltpu.VMEM((npt, 32), jnp.bfloat16),
            pltpu.VMEM((npt,), jnp.float32),
            pltpu.VMEM((2, 32), jnp.float32),
            pltpu.VMEM_SHARED((ntab, 32), jnp.bfloat16),
            pltpu.SemaphoreType.DMA,
            pltpu.SemaphoreType.DMA,
        ],
        compiler_params=_sc_params(),
    )
    def k(tab0_hbm, tab1_hbm, yh0_hbm, yh1_hbm, idx_hbm, w3_hbm, out_hbm,
          idx_v, rows_v, y_v, gp_v, w3_v, tab_sh, sem0, sem1):
        cid = lax.axis_index("c")
        sid = lax.axis_index("s")
        idx_row0 = sid * (nchunk * 4)
        sems = (sem0, sem1)

        for cc, tab_h, y_h in ((0, tab0_hbm, yh0_hbm), (1, tab1_hbm, yh1_hbm)):
            @pl.when(cid == cc)
            def _():
                pltpu.sync_copy(tab_h.at[pl.ds(sid * tpw, tpw)],
                                tab_sh.at[pl.ds(sid * tpw, tpw)])

            @pl.when(jnp.logical_and(cid == cc, sid < yfull))
            def _():
                pltpu.sync_copy(y_h.at[pl.ds(sid * npt, npt)], y_v)

            if ytail > 0:
                @pl.when(jnp.logical_and(cid == cc, sid == yfull))
                def _():
                    pltpu.sync_copy(y_h.at[pl.ds(yfull * npt, ytail)],
                                    y_v.at[pl.ds(0, ytail)])

        pltpu.sync_copy(w3_hbm, w3_v)
        plsc.subcore_barrier()
        w3a = w3_v[cid, pl.ds(0, 16)]
        w3b = w3_v[cid, pl.ds(16, 16)]
        lanes_i = lax.iota(jnp.int32, 16)

        def fire(buf, m):
            pltpu.sync_copy(idx_hbm.at[pl.ds(idx_row0 + m * 4, 4)],
                            idx_v.at[buf])
            for j in range(4):
                pltpu.async_copy(tab_sh.at[idx_v.at[buf].at[j]],
                                 rows_v.at[buf].at[pl.ds(j * 128, 128)],
                                 sems[buf])

        def drain(buf):
            pltpu.make_async_copy(tab0_hbm.at[pl.ds(0, 512)],
                                  rows_v.at[buf], sems[buf]).wait()

        fire(0, 0)

        @pl.loop(0, nchunk // 2)
        def _(m2):
            for buf in range(2):
                m = m2 * 2 + buf
                drain(buf)
                if buf == 0:
                    fire(1, m + 1)
                else:
                    @pl.when(m2 < nchunk // 2 - 1)
                    def _():
                        fire(0, m + 1)

                @pl.loop(0, 2)
                def _(hh):
                    acc = jnp.zeros((16,), jnp.float32)
                    for w16 in range(16):
                        w = hh * 16 + w16
                        evs, ods = [], []
                        for kk in range(16):
                            x = rows_v[buf, w * 16 + kk, :]
                            e, o = plsc.unpack(
                                x, format=plsc.PackFormat.INTERLEAVED)
                            evs.append(e)
                            ods.append(o)
                        while len(evs) > 1:
                            evs = [evs[2 * i] + evs[2 * i + 1]
                                   for i in range(len(evs) // 2)]
                            ods = [ods[2 * i] + ods[2 * i + 1]
                                   for i in range(len(ods) // 2)]
                        ye, yo = plsc.unpack(
                            y_v[m * 32 + w, :],
                            format=plsc.PackFormat.INTERLEAVED)
                        fe = jnp.maximum(
                            ye.astype(jnp.float32) - evs[0], 0.0)
                        fo = jnp.maximum(
                            yo.astype(jnp.float32) - ods[0], 0.0)
                        s = jnp.sum(fe * w3a + fo * w3b)
                        acc = jnp.where(lanes_i == w16, s, acc)
                    gp_v[pl.ds(m * 32 + hh * 16, 16)] = acc

        pltpu.sync_copy(gp_v, out_hbm.at[cid].at[pl.ds(sid * npt, npt)])

    return k(tab0, tab1, yh0, yh1, idx2, w3eo)


def _dense_pre(r80, wc, bc, npad, bn):
    def body(r_ref, w_ref, b_ref, o1, o2, o3):
        c = jnp.dot(r_ref[...], w_ref[...],
                    preferred_element_type=jnp.float32,
                    precision=lax.Precision.DEFAULT)
        c = c + 16.0 * b_ref[...]
        o1[...] = c[:, 0:64]
        o2[...] = c[:, 64:128]
        o3[...] = c[:, 128:129]

    return pl.pallas_call(
        body,
        grid=(npad // bn,),
        in_specs=[pl.BlockSpec((bn, 80), lambda i: (i, 0)),
                  pl.BlockSpec((80, 129), lambda i: (0, 0)),
                  pl.BlockSpec((1, 129), lambda i: (0, 0))],
        out_specs=[pl.BlockSpec((bn, 64), lambda i: (i, 0)),
                   pl.BlockSpec((bn, 64), lambda i: (i, 0)),
                   pl.BlockSpec((bn, 1), lambda i: (i, 0))],
        out_shape=[jax.ShapeDtypeStruct((npad, 64), jnp.float32),
                   jax.ShapeDtypeStruct((npad, 64), jnp.float32),
                   jax.ShapeDtypeStruct((npad, 1), jnp.float32)],
    )(r80, wc, bc)


def _dense_l1(c1, c2, d1, w1d, w2d, npad, bn):
    def body(c1_ref, c2_ref, d1_ref, w1_ref, w2_ref,
             t0_ref, t1_ref, y0_ref, y1_ref):
        d1v = d1_ref[...].reshape(bn, 1)
        f1 = jnp.maximum(c1_ref[...] + d1v * w1_ref[...], 0.0)
        t1 = jnp.dot(f1, w2_ref[...],
                     preferred_element_type=jnp.float32,
                     precision=lax.Precision.DEFAULT)
        th = t1.astype(jnp.bfloat16)
        t0_ref[...] = th[:, 0:32]
        t1_ref[...] = th[:, 32:64]
        yh = (c2_ref[...] + 16.0 * t1).astype(jnp.bfloat16)
        y0_ref[...] = yh[:, 0:32]
        y1_ref[...] = yh[:, 32:64]

    return pl.pallas_call(
        body,
        grid=(npad // bn,),
        in_specs=[pl.BlockSpec((bn, 64), lambda i: (i, 0)),
                  pl.BlockSpec((bn, 64), lambda i: (i, 0)),
                  pl.BlockSpec((bn,), lambda i: (i,)),
                  pl.BlockSpec((1, 64), lambda i: (0, 0)),
                  pl.BlockSpec((64, 64), lambda i: (0, 0))],
        out_specs=[pl.BlockSpec((bn, 32), lambda i: (i, 0)),
                   pl.BlockSpec((bn, 32), lambda i: (i, 0)),
                   pl.BlockSpec((bn, 32), lambda i: (i, 0)),
                   pl.BlockSpec((bn, 32), lambda i: (i, 0))],
        out_shape=[jax.ShapeDtypeStruct((npad, 32), jnp.bfloat16),
                   jax.ShapeDtypeStruct((npad, 32), jnp.bfloat16),
                   jax.ShapeDtypeStruct((npad, 32), jnp.bfloat16),
                   jax.ShapeDtypeStruct((npad, 32), jnp.bfloat16)],
    )(c1, c2, d1, w1d, w2d)


def _softmax_out(c3, d3, rows, cols):
    def body(c3_ref, d3_ref, o_ref):
        x = c3_ref[...] + d3_ref[...]
        e = jnp.exp(x - jnp.max(x))
        o_ref[...] = e / jnp.sum(e)

    return pl.pallas_call(
        body,
        out_shape=jax.ShapeDtypeStruct((rows, cols), jnp.float32),
    )(c3, d3)


def kernel(p_init, r_matrix, indices_neigh_tri, W1, b1, W2, b2, W3, b3):
    n, kp1 = indices_neigh_tri.shape
    kk = kp1 - 1
    r = r_matrix.shape[2]
    h = W1.shape[1]
    assert kk == 16 and r == 5 and h == 64
    npad = ((n + 2047) // 2048) * 2048
    bn = 2048

    neigh = indices_neigh_tri[:, 1:].astype(jnp.int32)
    neigh_p = jnp.pad(neigh, ((0, npad - n), (0, 0)))
    idx2 = neigh_p.reshape(-1, 128)
    idx_flat = neigh_p.reshape(-1)

    r80 = r_matrix.reshape(n, kk * r)
    wc = jnp.concatenate([jnp.tile(W1[:r], (kk, 1)),
                          jnp.tile(W2[:r], (kk, 1)),
                          jnp.tile(W3[:r], (kk, 1))], axis=1)
    bc = jnp.concatenate([b1, b2, b3])[None, :]

    c1, c2, c3 = _dense_pre(r80, wc, bc, npad, bn)
    p_pad = jnp.pad(p_init, (0, npad - n))
    d1 = _scalar_gather_diff(p_pad, idx_flat, npad)
    th0, th1, yh0, yh1 = _dense_l1(c1, c2, d1, W1[r:r + 1], W2[r:],
                                   npad, bn)
    w3f = W3[r:, 0]
    w3eo = jnp.stack([
        jnp.concatenate([w3f[0:32:2], w3f[1:32:2]]),
        jnp.concatenate([w3f[32:64:2], w3f[33:64:2]])])
    gp = _row_gather_partial(th0, th1, yh0, yh1, idx2, w3eo, npad)
    g = gp[0] + gp[1]
    d3 = _scalar_gather_diff(g, idx_flat, npad)[:n]
    rows, cols = 400, n // 400
    return _softmax_out(c3[:n].reshape(rows, cols), d3.reshape(rows, cols),
                        rows, cols).reshape(n)
